# baseline (device time: 66063 ns/iter reference)
import functools

import jax
import jax.numpy as jnp
from jax import lax
from jax.experimental import pallas as pl
from jax.experimental.pallas import tpu as pltpu

N_DEV = 4
B_LOC = 2
SQ = 128
D = 512
H_LOC = 8
DH = 64
SCALE = 0.125


def kernel(x, Wq, Wo, Wk, Wv):
    def body(x_ref, wq_ref, wo_ref, wk_ref, wv_ref, out_ref,
             comm_ref, part_ref, rs_ref, attn_ref,
             ag_send, ag_recv, rs_send, rs_recv):
        my = lax.axis_index("i")
        left = (my + N_DEV - 1) % N_DEV
        right = (my + 1) % N_DEV

        barrier = pltpu.get_barrier_semaphore()
        for nbr in (left, right):
            pl.semaphore_signal(barrier, inc=1, device_id=(nbr,),
                                device_id_type=pl.DeviceIdType.MESH)
        pl.semaphore_wait(barrier, 2)

        wq = wq_ref[...].astype(jnp.bfloat16)
        wk = wk_ref[...].astype(jnp.bfloat16)
        wv = wv_ref[...].astype(jnp.bfloat16)
        wo = wo_ref[...].astype(jnp.bfloat16)

        def compute_chunk(r, slot):
            for b in range(B_LOC):
                x_b = comm_ref[slot, b]
                q = lax.dot(x_b, wq,
                            preferred_element_type=jnp.float32)
                k = lax.dot(x_b, wk,
                            preferred_element_type=jnp.float32)
                v = lax.dot(x_b, wv,
                            preferred_element_type=jnp.float32)
                qb = q.astype(jnp.bfloat16)
                kb = k.astype(jnp.bfloat16)
                vb = v.astype(jnp.bfloat16)
                for h in range(H_LOC):
                    sl = slice(h * DH, (h + 1) * DH)
                    s = lax.dot_general(
                        qb[:, sl], kb[:, sl],
                        dimension_numbers=(((1,), (1,)), ((), ())),
                        preferred_element_type=jnp.float32,
                    ) * SCALE
                    m = jnp.max(s, axis=1, keepdims=True)
                    e = jnp.exp(s - m)
                    l = jnp.sum(e, axis=1, keepdims=True)
                    o = lax.dot(e.astype(jnp.bfloat16), vb[:, sl],
                                preferred_element_type=jnp.float32)
                    attn_ref[:, sl] = (o / l).astype(jnp.bfloat16)
                part_ref[r, b] = lax.dot(attn_ref[...], wo,
                                         preferred_element_type=jnp.float32)

        comm_ref[3] = x_ref[...].astype(jnp.bfloat16)
        for h in range(N_DEV - 1):
            src_slot = 3 if h == 0 else h - 1
            rdma = pltpu.make_async_remote_copy(
                src_ref=comm_ref.at[src_slot],
                dst_ref=comm_ref.at[h],
                send_sem=ag_send.at[h],
                recv_sem=ag_recv.at[h],
                device_id=(right,),
                device_id_type=pl.DeviceIdType.MESH,
            )
            rdma.start()
            rdma.wait()

        compute_chunk(0, 3)
        for h in range(N_DEV - 1):
            compute_chunk(3 - h, h)

        for s in range(N_DEV - 1):
            if s > 0:
                part_ref[3 - s] = part_ref[3 - s] + rs_ref[s - 1]
            rdma = pltpu.make_async_remote_copy(
                src_ref=part_ref.at[3 - s],
                dst_ref=rs_ref.at[s],
                send_sem=rs_send.at[s],
                recv_sem=rs_recv.at[s],
                device_id=(right,),
                device_id_type=pl.DeviceIdType.MESH,
            )
            rdma.start()
            rdma.wait()

        out_ref[...] = part_ref[0] + rs_ref[N_DEV - 2]

        @functools.partial(pl.run_scoped,
                           second_barrier=pltpu.SemaphoreType.REGULAR)
        def _(second_barrier):
            for nbr in (left, right):
                pl.semaphore_signal(second_barrier, inc=1, device_id=(nbr,),
                                    device_id_type=pl.DeviceIdType.MESH)
            pl.semaphore_wait(second_barrier, 2)

    return pl.pallas_call(
        body,
        out_shape=jax.ShapeDtypeStruct((B_LOC, SQ, D), jnp.float32),
        in_specs=[pl.BlockSpec(memory_space=pltpu.VMEM)] * 5,
        out_specs=pl.BlockSpec(memory_space=pltpu.VMEM),
        scratch_shapes=[
            pltpu.VMEM((N_DEV, B_LOC, SQ, D), jnp.bfloat16),
            pltpu.VMEM((N_DEV, B_LOC, SQ, D), jnp.float32),
            pltpu.VMEM((N_DEV - 1, B_LOC, SQ, D), jnp.float32),
            pltpu.VMEM((SQ, D), jnp.bfloat16),
            pltpu.SemaphoreType.DMA((N_DEV - 1,)),
            pltpu.SemaphoreType.DMA((N_DEV - 1,)),
            pltpu.SemaphoreType.DMA((N_DEV - 1,)),
            pltpu.SemaphoreType.DMA((N_DEV - 1,)),
        ],
        compiler_params=pltpu.CompilerParams(collective_id=0),
    )(x, Wq, Wo, Wk, Wv)


# device time: 37882 ns/iter; 1.7439x vs baseline; 1.7439x over previous
import functools

import jax
import jax.numpy as jnp
from jax import lax
from jax.experimental import pallas as pl
from jax.experimental.pallas import tpu as pltpu

N_DEV = 4
B_LOC = 2
SQ = 128
D = 512
H_LOC = 8
DH = 64
SCALE = 0.125


def kernel(x, Wq, Wo, Wk, Wv):
    def body(x_ref, wq_ref, wo_ref, wk_ref, wv_ref, out_ref,
             comm_ref, part_ref, rs_ref, attn_ref,
             ag_send, ag_recv, rs_send, rs_recv):
        my = lax.axis_index("i")
        left = (my + N_DEV - 1) % N_DEV
        right = (my + 1) % N_DEV

        barrier = pltpu.get_barrier_semaphore()
        for nbr in (left, right):
            pl.semaphore_signal(barrier, inc=1, device_id=(nbr,),
                                device_id_type=pl.DeviceIdType.MESH)
        pl.semaphore_wait(barrier, 2)

        def ag_hop(h):
            return pltpu.make_async_remote_copy(
                src_ref=comm_ref.at[3 if h == 0 else h - 1],
                dst_ref=comm_ref.at[h],
                send_sem=ag_send.at[h],
                recv_sem=ag_recv.at[h],
                device_id=(right,),
                device_id_type=pl.DeviceIdType.MESH,
            )

        def rs_hop(s):
            return pltpu.make_async_remote_copy(
                src_ref=part_ref.at[3 - s],
                dst_ref=rs_ref.at[s],
                send_sem=rs_send.at[s],
                recv_sem=rs_recv.at[s],
                device_id=(right,),
                device_id_type=pl.DeviceIdType.MESH,
            )

        def compute_chunk(r, slot):
            for b in range(B_LOC):
                x_b = comm_ref[slot, b]
                q = lax.dot(x_b, wq, preferred_element_type=jnp.float32)
                k = lax.dot(x_b, wk, preferred_element_type=jnp.float32)
                v = lax.dot(x_b, wv, preferred_element_type=jnp.float32)
                qb = q.astype(jnp.bfloat16)
                kb = k.astype(jnp.bfloat16)
                vb = v.astype(jnp.bfloat16)
                for h in range(H_LOC):
                    sl = slice(h * DH, (h + 1) * DH)
                    s = lax.dot_general(
                        qb[:, sl], kb[:, sl],
                        dimension_numbers=(((1,), (1,)), ((), ())),
                        preferred_element_type=jnp.float32,
                    ) * SCALE
                    m = jnp.max(s, axis=1, keepdims=True)
                    e = jnp.exp(s - m)
                    l = jnp.sum(e, axis=1, keepdims=True)
                    o = lax.dot(e.astype(jnp.bfloat16), vb[:, sl],
                                preferred_element_type=jnp.float32)
                    attn_ref[:, sl] = (o / l).astype(jnp.bfloat16)
                part_ref[r, b] = lax.dot(
                    attn_ref[...], wo, preferred_element_type=jnp.float32
                ).astype(jnp.bfloat16)

        comm_ref[3] = x_ref[...].astype(jnp.bfloat16)
        ag0 = ag_hop(0)
        ag0.start()

        wq = wq_ref[...].astype(jnp.bfloat16)
        wk = wk_ref[...].astype(jnp.bfloat16)
        wv = wv_ref[...].astype(jnp.bfloat16)
        wo = wo_ref[...].astype(jnp.bfloat16)
        compute_chunk(0, 3)

        ag0.wait()
        ag1 = ag_hop(1)
        ag1.start()
        compute_chunk(3, 0)

        ag1.wait()
        ag2 = ag_hop(2)
        ag2.start()
        rs0 = rs_hop(0)
        rs0.start()
        compute_chunk(2, 1)

        ag2.wait()
        rs0.wait()
        part_ref[2] = part_ref[2] + rs_ref[0]
        rs1 = rs_hop(1)
        rs1.start()
        compute_chunk(1, 2)

        rs1.wait()
        part_ref[1] = part_ref[1] + rs_ref[1]
        rs2 = rs_hop(2)
        rs2.start()
        rs2.wait()
        out_ref[...] = (part_ref[0].astype(jnp.float32)
                        + rs_ref[2].astype(jnp.float32))

        @functools.partial(pl.run_scoped,
                           second_barrier=pltpu.SemaphoreType.REGULAR)
        def _(second_barrier):
            for nbr in (left, right):
                pl.semaphore_signal(second_barrier, inc=1, device_id=(nbr,),
                                    device_id_type=pl.DeviceIdType.MESH)
            pl.semaphore_wait(second_barrier, 2)

    return pl.pallas_call(
        body,
        out_shape=jax.ShapeDtypeStruct((B_LOC, SQ, D), jnp.float32),
        in_specs=[pl.BlockSpec(memory_space=pltpu.VMEM)] * 5,
        out_specs=pl.BlockSpec(memory_space=pltpu.VMEM),
        scratch_shapes=[
            pltpu.VMEM((N_DEV, B_LOC, SQ, D), jnp.bfloat16),
            pltpu.VMEM((N_DEV, B_LOC, SQ, D), jnp.bfloat16),
            pltpu.VMEM((N_DEV - 1, B_LOC, SQ, D), jnp.bfloat16),
            pltpu.VMEM((SQ, D), jnp.bfloat16),
            pltpu.SemaphoreType.DMA((N_DEV - 1,)),
            pltpu.SemaphoreType.DMA((N_DEV - 1,)),
            pltpu.SemaphoreType.DMA((N_DEV - 1,)),
            pltpu.SemaphoreType.DMA((N_DEV - 1,)),
        ],
        compiler_params=pltpu.CompilerParams(collective_id=0),
    )(x, Wq, Wo, Wk, Wv)


# device time: 35880 ns/iter; 1.8412x vs baseline; 1.0558x over previous
import functools

import jax
import jax.numpy as jnp
from jax import lax
from jax.experimental import pallas as pl
from jax.experimental.pallas import tpu as pltpu

N_DEV = 4
B_LOC = 2
SQ = 128
D = 512
H_LOC = 8
DH = 64
SCALE = 0.125
BH = B_LOC * H_LOC


def kernel(x, Wq, Wo, Wk, Wv):
    def body(x_ref, wq_ref, wo_ref, wk_ref, wv_ref, out_ref,
             comm_ref, part_ref, rs_ref, wqkv_ref,
             q4_ref, k4_ref, v4_ref, attn_ref,
             ag_send, ag_recv, rs_send, rs_recv):
        my = lax.axis_index("i")
        left = (my + N_DEV - 1) % N_DEV
        right = (my + 1) % N_DEV

        barrier = pltpu.get_barrier_semaphore()
        for nbr in (left, right):
            pl.semaphore_signal(barrier, inc=1, device_id=(nbr,),
                                device_id_type=pl.DeviceIdType.MESH)
        pl.semaphore_wait(barrier, 2)

        def ag_hop(h):
            return pltpu.make_async_remote_copy(
                src_ref=comm_ref.at[3 if h == 0 else h - 1],
                dst_ref=comm_ref.at[h],
                send_sem=ag_send.at[h],
                recv_sem=ag_recv.at[h],
                device_id=(right,),
                device_id_type=pl.DeviceIdType.MESH,
            )

        def rs_hop(s):
            return pltpu.make_async_remote_copy(
                src_ref=part_ref.at[3 - s],
                dst_ref=rs_ref.at[s],
                send_sem=rs_send.at[s],
                recv_sem=rs_recv.at[s],
                device_id=(right,),
                device_id_type=pl.DeviceIdType.MESH,
            )

        def compute_chunk(r, slot):
            x2 = comm_ref[slot].reshape(B_LOC * SQ, D)
            qkv = lax.dot(x2, wqkv_ref[...],
                          preferred_element_type=jnp.float32
                          ).astype(jnp.bfloat16)
            for b in range(B_LOC):
                rows = slice(b * SQ, (b + 1) * SQ)
                for h in range(H_LOC):
                    i = b * H_LOC + h
                    q4_ref[i] = qkv[rows, 0 * D + h * DH:0 * D + (h + 1) * DH]
                    k4_ref[i] = qkv[rows, 1 * D + h * DH:1 * D + (h + 1) * DH]
                    v4_ref[i] = qkv[rows, 2 * D + h * DH:2 * D + (h + 1) * DH]
            s = lax.dot_general(
                q4_ref[...], k4_ref[...],
                dimension_numbers=(((2,), (2,)), ((0,), (0,))),
                preferred_element_type=jnp.float32,
            )
            e = jnp.exp(s)
            l = jnp.sum(e, axis=2, keepdims=True)
            o = lax.dot_general(
                e.astype(jnp.bfloat16), v4_ref[...],
                dimension_numbers=(((2,), (1,)), ((0,), (0,))),
                preferred_element_type=jnp.float32,
            )
            ob = (o * (1.0 / l)).astype(jnp.bfloat16)
            for b in range(B_LOC):
                rows = slice(b * SQ, (b + 1) * SQ)
                for h in range(H_LOC):
                    attn_ref[rows, h * DH:(h + 1) * DH] = ob[b * H_LOC + h]
            part_ref[r] = lax.dot(
                attn_ref[...], wo, preferred_element_type=jnp.float32
            ).astype(jnp.bfloat16).reshape(B_LOC, SQ, D)

        comm_ref[3] = x_ref[...].astype(jnp.bfloat16)
        ag0 = ag_hop(0)
        ag0.start()

        wqkv_ref[:, 0 * D:1 * D] = (wq_ref[...] * SCALE).astype(jnp.bfloat16)
        wqkv_ref[:, 1 * D:2 * D] = wk_ref[...].astype(jnp.bfloat16)
        wqkv_ref[:, 2 * D:3 * D] = wv_ref[...].astype(jnp.bfloat16)
        wo = wo_ref[...].astype(jnp.bfloat16)
        compute_chunk(0, 3)

        ag0.wait()
        ag1 = ag_hop(1)
        ag1.start()
        compute_chunk(3, 0)

        ag1.wait()
        ag2 = ag_hop(2)
        ag2.start()
        rs0 = rs_hop(0)
        rs0.start()
        compute_chunk(2, 1)

        ag2.wait()
        rs0.wait()
        part_ref[2] = part_ref[2] + rs_ref[0]
        rs1 = rs_hop(1)
        rs1.start()
        compute_chunk(1, 2)

        rs1.wait()
        part_ref[1] = part_ref[1] + rs_ref[1]
        rs2 = rs_hop(2)
        rs2.start()
        rs2.wait()
        out_ref[...] = (part_ref[0].astype(jnp.float32)
                        + rs_ref[2].astype(jnp.float32))

        @functools.partial(pl.run_scoped,
                           second_barrier=pltpu.SemaphoreType.REGULAR)
        def _(second_barrier):
            for nbr in (left, right):
                pl.semaphore_signal(second_barrier, inc=1, device_id=(nbr,),
                                    device_id_type=pl.DeviceIdType.MESH)
            pl.semaphore_wait(second_barrier, 2)

    return pl.pallas_call(
        body,
        out_shape=jax.ShapeDtypeStruct((B_LOC, SQ, D), jnp.float32),
        in_specs=[pl.BlockSpec(memory_space=pltpu.VMEM)] * 5,
        out_specs=pl.BlockSpec(memory_space=pltpu.VMEM),
        scratch_shapes=[
            pltpu.VMEM((N_DEV, B_LOC, SQ, D), jnp.bfloat16),
            pltpu.VMEM((N_DEV, B_LOC, SQ, D), jnp.bfloat16),
            pltpu.VMEM((N_DEV - 1, B_LOC, SQ, D), jnp.bfloat16),
            pltpu.VMEM((D, 3 * D), jnp.bfloat16),
            pltpu.VMEM((BH, SQ, DH), jnp.bfloat16),
            pltpu.VMEM((BH, SQ, DH), jnp.bfloat16),
            pltpu.VMEM((BH, SQ, DH), jnp.bfloat16),
            pltpu.VMEM((B_LOC * SQ, D), jnp.bfloat16),
            pltpu.SemaphoreType.DMA((N_DEV - 1,)),
            pltpu.SemaphoreType.DMA((N_DEV - 1,)),
            pltpu.SemaphoreType.DMA((N_DEV - 1,)),
            pltpu.SemaphoreType.DMA((N_DEV - 1,)),
        ],
        compiler_params=pltpu.CompilerParams(collective_id=0),
    )(x, Wq, Wo, Wk, Wv)


# device time: 25990 ns/iter; 2.5419x vs baseline; 1.3805x over previous
import jax
import jax.numpy as jnp
from jax import lax
from jax.experimental import pallas as pl
from jax.experimental.pallas import tpu as pltpu

N_DEV = 4
B_LOC = 2
SQ = 128
D = 512
H_LOC = 8
DH = 64
SCALE = 0.125
BH = B_LOC * H_LOC


def kernel(x, Wq, Wo, Wk, Wv):
    def body(x_ref, wq_ref, wo_ref, wk_ref, wv_ref, out_ref,
             comm_ref, part_ref, rsbuf_ref, wqkv_ref,
             q4_ref, k4_ref, v4_ref, attn_ref,
             ag_send, ag_recv, rs_send, rs_recv):
        my = lax.axis_index("i")
        left = (my + N_DEV - 1) % N_DEV
        right = (my + 1) % N_DEV
        diag = (my + 2) % N_DEV

        barrier = pltpu.get_barrier_semaphore()
        for nbr in (left, right, diag):
            pl.semaphore_signal(barrier, inc=1, device_id=(nbr,),
                                device_id_type=pl.DeviceIdType.MESH)
        pl.semaphore_wait(barrier, 3)

        def ag_send_to(dst_dev, slot, sem_i):
            return pltpu.make_async_remote_copy(
                src_ref=comm_ref.at[0],
                dst_ref=comm_ref.at[slot],
                send_sem=ag_send.at[sem_i],
                recv_sem=ag_recv.at[slot],
                device_id=(dst_dev,),
                device_id_type=pl.DeviceIdType.MESH,
            )

        def ag_recv_from(src_dev, slot):
            return pltpu.make_async_remote_copy(
                src_ref=comm_ref.at[0],
                dst_ref=comm_ref.at[slot],
                send_sem=ag_send.at[0],
                recv_sem=ag_recv.at[slot],
                device_id=(src_dev,),
                device_id_type=pl.DeviceIdType.MESH,
            )

        def rs_send_to(dst_dev, part_slot, buf_slot):
            return pltpu.make_async_remote_copy(
                src_ref=part_ref.at[part_slot],
                dst_ref=rsbuf_ref.at[buf_slot],
                send_sem=rs_send.at[buf_slot],
                recv_sem=rs_recv.at[buf_slot],
                device_id=(dst_dev,),
                device_id_type=pl.DeviceIdType.MESH,
            )

        def rs_recv_from(src_dev, buf_slot):
            return pltpu.make_async_remote_copy(
                src_ref=part_ref.at[0],
                dst_ref=rsbuf_ref.at[buf_slot],
                send_sem=rs_send.at[0],
                recv_sem=rs_recv.at[buf_slot],
                device_id=(src_dev,),
                device_id_type=pl.DeviceIdType.MESH,
            )

        def compute_chunk(r):
            x2 = comm_ref[r].reshape(B_LOC * SQ, D)
            qkv = lax.dot(x2, wqkv_ref[...],
                          preferred_element_type=jnp.float32
                          ).astype(jnp.bfloat16)
            for b in range(B_LOC):
                rows = slice(b * SQ, (b + 1) * SQ)
                for h in range(H_LOC):
                    i = b * H_LOC + h
                    q4_ref[i] = qkv[rows, 0 * D + h * DH:0 * D + (h + 1) * DH]
                    k4_ref[i] = qkv[rows, 1 * D + h * DH:1 * D + (h + 1) * DH]
                    v4_ref[i] = qkv[rows, 2 * D + h * DH:2 * D + (h + 1) * DH]
            s = lax.dot_general(
                q4_ref[...], k4_ref[...],
                dimension_numbers=(((2,), (2,)), ((0,), (0,))),
                preferred_element_type=jnp.float32,
            )
            e = jnp.exp(s)
            l = jnp.sum(e, axis=2, keepdims=True)
            o = lax.dot_general(
                e.astype(jnp.bfloat16), v4_ref[...],
                dimension_numbers=(((2,), (1,)), ((0,), (0,))),
                preferred_element_type=jnp.float32,
            )
            ob = (o * (1.0 / l)).astype(jnp.bfloat16)
            for b in range(B_LOC):
                rows = slice(b * SQ, (b + 1) * SQ)
                for h in range(H_LOC):
                    attn_ref[rows, h * DH:(h + 1) * DH] = ob[b * H_LOC + h]
            part_ref[r] = lax.dot(
                attn_ref[...], wo, preferred_element_type=jnp.float32
            ).astype(jnp.bfloat16).reshape(B_LOC, SQ, D)

        comm_ref[0] = x_ref[...].astype(jnp.bfloat16)
        snd_l = ag_send_to(left, 1, 0)
        snd_r = ag_send_to(right, 3, 1)
        snd_d = ag_send_to(diag, 2, 2)
        snd_l.start()
        snd_r.start()
        snd_d.start()

        wqkv_ref[:, 0 * D:1 * D] = (wq_ref[...] * SCALE).astype(jnp.bfloat16)
        wqkv_ref[:, 1 * D:2 * D] = wk_ref[...].astype(jnp.bfloat16)
        wqkv_ref[:, 2 * D:3 * D] = wv_ref[...].astype(jnp.bfloat16)
        wo = wo_ref[...].astype(jnp.bfloat16)
        compute_chunk(0)

        ag_recv_from(left, 3).wait_recv()
        compute_chunk(3)
        rs_l = rs_send_to(left, 3, 1)
        rs_l.start()

        ag_recv_from(right, 1).wait_recv()
        compute_chunk(1)
        rs_r = rs_send_to(right, 1, 0)
        rs_r.start()

        ag_recv_from(diag, 2).wait_recv()
        compute_chunk(2)
        rs_d = rs_send_to(diag, 2, 2)
        rs_d.start()

        rs_recv_from(left, 0).wait_recv()
        acc = (part_ref[0].astype(jnp.float32)
               + rsbuf_ref[0].astype(jnp.float32))
        rs_recv_from(right, 1).wait_recv()
        acc = acc + rsbuf_ref[1].astype(jnp.float32)
        rs_recv_from(diag, 2).wait_recv()
        out_ref[...] = acc + rsbuf_ref[2].astype(jnp.float32)

        for snd in (snd_l, snd_r, snd_d, rs_l, rs_r, rs_d):
            snd.wait_send()

    return pl.pallas_call(
        body,
        out_shape=jax.ShapeDtypeStruct((B_LOC, SQ, D), jnp.float32),
        in_specs=[pl.BlockSpec(memory_space=pltpu.VMEM)] * 5,
        out_specs=pl.BlockSpec(memory_space=pltpu.VMEM),
        scratch_shapes=[
            pltpu.VMEM((N_DEV, B_LOC, SQ, D), jnp.bfloat16),
            pltpu.VMEM((N_DEV, B_LOC, SQ, D), jnp.bfloat16),
            pltpu.VMEM((3, B_LOC, SQ, D), jnp.bfloat16),
            pltpu.VMEM((D, 3 * D), jnp.bfloat16),
            pltpu.VMEM((BH, SQ, DH), jnp.bfloat16),
            pltpu.VMEM((BH, SQ, DH), jnp.bfloat16),
            pltpu.VMEM((BH, SQ, DH), jnp.bfloat16),
            pltpu.VMEM((B_LOC * SQ, D), jnp.bfloat16),
            pltpu.SemaphoreType.DMA((3,)),
            pltpu.SemaphoreType.DMA((N_DEV,)),
            pltpu.SemaphoreType.DMA((3,)),
            pltpu.SemaphoreType.DMA((3,)),
        ],
        compiler_params=pltpu.CompilerParams(collective_id=0),
    )(x, Wq, Wo, Wk, Wv)
